# baseline (speedup 1.0000x reference)
import functools

import jax
import jax.numpy as jnp
from jax import lax
from jax.experimental import pallas as pl
from jax.experimental.pallas import tpu as pltpu
from jax.experimental.pallas import tpu_sc as plsc

DIM = 128
NUM_CORES = 2
NUM_SUBCORES = 16
NUM_WORKERS = NUM_CORES * NUM_SUBCORES
CHUNK = 128


def _emb_body(seq_len, rows_per_worker, ids_hbm, word_hbm, pos_hbm, out_hbm,
              idx_v, rows_v, sem_idx, sem_pos, sem_gat, sem_out):
    n_chunks = rows_per_worker // CHUNK
    wid = lax.axis_index("s") * NUM_CORES + lax.axis_index("c")
    base = wid * rows_per_worker
    workers_per_batch = seq_len // rows_per_worker
    b = wid // workers_per_batch
    s_base = lax.rem(base, seq_len)
    idx_cp = pltpu.async_copy(ids_hbm.at[b, pl.ds(s_base, rows_per_worker)],
                              idx_v, sem_idx)
    pos_cps = [
        pltpu.async_copy(pos_hbm.at[pl.ds(s_base + j * CHUNK, CHUNK)],
                         rows_v.at[pl.ds(j * CHUNK, CHUNK)], sem_pos.at[j])
        for j in range(n_chunks)
    ]
    idx_cp.wait()
    gat_cps = []
    for j in range(n_chunks):
        pos_cps[j].wait()
        gat_cps.append(pltpu.async_copy(
            word_hbm.at[idx_v.at[pl.ds(j * CHUNK, CHUNK)]],
            rows_v.at[pl.ds(j * CHUNK, CHUNK)], sem_gat.at[j], add=True))
    out_cps = []
    for j in range(n_chunks):
        gat_cps[j].wait()
        out_cps.append(pltpu.async_copy(
            rows_v.at[pl.ds(j * CHUNK, CHUNK)],
            out_hbm.at[pl.ds(base + j * CHUNK, CHUNK)], sem_out.at[j]))
    for cp in out_cps:
        cp.wait()


def kernel(input_ids, word_embeddings, position_embeddings):
    batch, seq_len = input_ids.shape
    total = batch * seq_len
    rows_per_worker = total // NUM_WORKERS
    n_chunks = rows_per_worker // CHUNK
    mesh = plsc.VectorSubcoreMesh(core_axis_name="c", subcore_axis_name="s")
    body = functools.partial(_emb_body, seq_len, rows_per_worker)
    out = pl.kernel(
        body,
        mesh=mesh,
        out_type=jax.ShapeDtypeStruct((total, DIM), jnp.float32),
        scratch_types=[
            pltpu.VMEM((rows_per_worker,), jnp.int32),
            pltpu.VMEM((rows_per_worker, DIM), jnp.float32),
            pltpu.SemaphoreType.DMA,
            pltpu.SemaphoreType.DMA((n_chunks,)),
            pltpu.SemaphoreType.DMA((n_chunks,)),
            pltpu.SemaphoreType.DMA((n_chunks,)),
        ],
    )(input_ids, word_embeddings, position_embeddings)
    return out.reshape(batch, seq_len, DIM)


# speedup vs baseline: 1.0137x; 1.0137x over previous
import functools

import jax
import jax.numpy as jnp
from jax import lax
from jax.experimental import pallas as pl
from jax.experimental.pallas import tpu as pltpu
from jax.experimental.pallas import tpu_sc as plsc

DIM = 128
NUM_CORES = 2
NUM_SUBCORES = 16
NUM_WORKERS = NUM_CORES * NUM_SUBCORES
CHUNK = 32


def _emb_body(seq_len, rows_per_worker, ids_hbm, word_hbm, pos_hbm, out_hbm,
              idx_v, rows_v, sem_idx, sem_pos, sem_gat, sem_out):
    n_chunks = rows_per_worker // CHUNK
    wid = lax.axis_index("s") * NUM_CORES + lax.axis_index("c")
    base = wid * rows_per_worker
    workers_per_batch = seq_len // rows_per_worker
    b = wid // workers_per_batch
    s_base = lax.rem(base, seq_len)
    idx_cp = pltpu.async_copy(ids_hbm.at[b, pl.ds(s_base, rows_per_worker)],
                              idx_v, sem_idx)
    pos_cps = [
        pltpu.async_copy(pos_hbm.at[pl.ds(s_base + j * CHUNK, CHUNK)],
                         rows_v.at[pl.ds(j * CHUNK, CHUNK)], sem_pos.at[j])
        for j in range(n_chunks)
    ]
    idx_cp.wait()
    gat_cps = []
    for j in range(n_chunks):
        pos_cps[j].wait()
        gat_cps.append(pltpu.async_copy(
            word_hbm.at[idx_v.at[pl.ds(j * CHUNK, CHUNK)]],
            rows_v.at[pl.ds(j * CHUNK, CHUNK)], sem_gat.at[j], add=True))
    out_cps = []
    for j in range(n_chunks):
        gat_cps[j].wait()
        out_cps.append(pltpu.async_copy(
            rows_v.at[pl.ds(j * CHUNK, CHUNK)],
            out_hbm.at[pl.ds(base + j * CHUNK, CHUNK)], sem_out.at[j]))
    for cp in out_cps:
        cp.wait()


def kernel(input_ids, word_embeddings, position_embeddings):
    batch, seq_len = input_ids.shape
    total = batch * seq_len
    rows_per_worker = total // NUM_WORKERS
    n_chunks = rows_per_worker // CHUNK
    mesh = plsc.VectorSubcoreMesh(core_axis_name="c", subcore_axis_name="s")
    body = functools.partial(_emb_body, seq_len, rows_per_worker)
    out = pl.kernel(
        body,
        mesh=mesh,
        out_type=jax.ShapeDtypeStruct((total, DIM), jnp.float32),
        scratch_types=[
            pltpu.VMEM((rows_per_worker,), jnp.int32),
            pltpu.VMEM((rows_per_worker, DIM), jnp.float32),
            pltpu.SemaphoreType.DMA,
            pltpu.SemaphoreType.DMA((n_chunks,)),
            pltpu.SemaphoreType.DMA((n_chunks,)),
            pltpu.SemaphoreType.DMA((n_chunks,)),
        ],
    )(input_ids, word_embeddings, position_embeddings)
    return out.reshape(batch, seq_len, DIM)


# speedup vs baseline: 1.0276x; 1.0137x over previous
import functools

import jax
import jax.numpy as jnp
from jax import lax
from jax.experimental import pallas as pl
from jax.experimental.pallas import tpu as pltpu
from jax.experimental.pallas import tpu_sc as plsc

DIM = 128
NUM_CORES = 2
NUM_SUBCORES = 16
NUM_WORKERS = NUM_CORES * NUM_SUBCORES
LANES = 16


def _emb_body(batch, seq_len, ids_hbm, word_hbm, pos_hbm, out_hbm,
              idx_v, pidx_v, rows_v, pos_sh,
              sem_idx, sem_pos, sem_gat, sem_add, sem_out):
    span = seq_len // NUM_WORKERS
    cid = lax.axis_index("c")
    sid = lax.axis_index("s")
    wid = sid * NUM_CORES + cid
    s0 = wid * span
    sh_base = sid * span
    pos_cp = pltpu.async_copy(pos_hbm.at[pl.ds(s0, span)],
                              pos_sh.at[pl.ds(sh_base, span)], sem_pos)
    idx_cps = [
        pltpu.async_copy(ids_hbm.at[b, pl.ds(s0, span)], idx_v.at[b],
                         sem_idx.at[b])
        for b in range(batch)
    ]
    for k in range(span // LANES):
        pidx_v[pl.ds(k * LANES, LANES)] = (
            lax.iota(jnp.int32, LANES) + (sh_base + k * LANES))
    gat_cps = []
    for b in range(batch):
        idx_cps[b].wait()
        gat_cps.append(pltpu.async_copy(
            word_hbm.at[idx_v.at[b]],
            rows_v.at[pl.ds(b * span, span)], sem_gat.at[b]))
    pos_cp.wait()
    add_cps = []
    for b in range(batch):
        gat_cps[b].wait()
        add_cps.append(pltpu.async_copy(
            pos_sh.at[pidx_v],
            rows_v.at[pl.ds(b * span, span)], sem_add.at[b], add=True))
    out_cps = []
    for b in range(batch):
        add_cps[b].wait()
        out_cps.append(pltpu.async_copy(
            rows_v.at[pl.ds(b * span, span)],
            out_hbm.at[pl.ds(b * seq_len + s0, span)], sem_out.at[b]))
    for cp in out_cps:
        cp.wait()


def kernel(input_ids, word_embeddings, position_embeddings):
    batch, seq_len = input_ids.shape
    span = seq_len // NUM_WORKERS
    mesh = plsc.VectorSubcoreMesh(core_axis_name="c", subcore_axis_name="s")
    body = functools.partial(_emb_body, batch, seq_len)
    out = pl.kernel(
        body,
        mesh=mesh,
        out_type=jax.ShapeDtypeStruct((batch * seq_len, DIM), jnp.float32),
        scratch_types=[
            pltpu.VMEM((batch, span), jnp.int32),
            pltpu.VMEM((span,), jnp.int32),
            pltpu.VMEM((batch * span, DIM), jnp.float32),
            pltpu.VMEM_SHARED((NUM_SUBCORES * span, DIM), jnp.float32),
            pltpu.SemaphoreType.DMA((batch,)),
            pltpu.SemaphoreType.DMA,
            pltpu.SemaphoreType.DMA((batch,)),
            pltpu.SemaphoreType.DMA((batch,)),
            pltpu.SemaphoreType.DMA((batch,)),
        ],
    )(input_ids, word_embeddings, position_embeddings)
    return out.reshape(batch, seq_len, DIM)


# speedup vs baseline: 1.0383x; 1.0104x over previous
import functools

import jax
import jax.numpy as jnp
from jax import lax
from jax.experimental import pallas as pl
from jax.experimental.pallas import tpu as pltpu
from jax.experimental.pallas import tpu_sc as plsc

DIM = 128
NUM_CORES = 2
NUM_SUBCORES = 16
NUM_WORKERS = NUM_CORES * NUM_SUBCORES
LANES = 16


def _emb_body(batch, seq_len, ids_hbm, word_hbm, pos_hbm, out_hbm,
              idx_v, pidx_v, rows_v, pos_sh,
              sem_idx, sem_pos, sem_gat, sem_add, sem_out):
    span = seq_len // NUM_WORKERS
    cid = lax.axis_index("c")
    sid = lax.axis_index("s")
    wid = sid * NUM_CORES + cid
    s0 = wid * span
    sh_base = sid * span
    ch = span // 2
    pos_cp = pltpu.async_copy(pos_hbm.at[pl.ds(s0, span)],
                              pos_sh.at[pl.ds(sh_base, span)], sem_pos)
    idx_cps = [
        pltpu.async_copy(ids_hbm.at[b, pl.ds(s0, span)], idx_v.at[b],
                         sem_idx.at[b])
        for b in range(batch)
    ]
    for k in range(span // LANES):
        pidx_v[pl.ds(k * LANES, LANES)] = (
            lax.iota(jnp.int32, LANES) + (sh_base + k * LANES))
    chunks = [(b, k) for b in range(batch) for k in range(span // ch)]
    for cp in idx_cps:
        cp.wait()
    gat_cps = []
    for i, (b, k) in enumerate(chunks):
        gat_cps.append(pltpu.async_copy(
            word_hbm.at[idx_v.at[b, pl.ds(k * ch, ch)]],
            rows_v.at[pl.ds(b * span + k * ch, ch)], sem_gat.at[i]))
    pos_cp.wait()
    add_cps = []
    for i, (b, k) in enumerate(chunks):
        gat_cps[i].wait()
        add_cps.append(pltpu.async_copy(
            pos_sh.at[pidx_v.at[pl.ds(k * ch, ch)]],
            rows_v.at[pl.ds(b * span + k * ch, ch)], sem_add.at[i], add=True))
    out_cps = []
    for i, (b, k) in enumerate(chunks):
        add_cps[i].wait()
        out_cps.append(pltpu.async_copy(
            rows_v.at[pl.ds(b * span + k * ch, ch)],
            out_hbm.at[pl.ds(b * seq_len + s0 + k * ch, ch)], sem_out.at[i]))
    for cp in out_cps:
        cp.wait()


def kernel(input_ids, word_embeddings, position_embeddings):
    batch, seq_len = input_ids.shape
    span = seq_len // NUM_WORKERS
    mesh = plsc.VectorSubcoreMesh(core_axis_name="c", subcore_axis_name="s")
    body = functools.partial(_emb_body, batch, seq_len)
    out = pl.kernel(
        body,
        mesh=mesh,
        out_type=jax.ShapeDtypeStruct((batch * seq_len, DIM), jnp.float32),
        scratch_types=[
            pltpu.VMEM((batch, span), jnp.int32),
            pltpu.VMEM((span,), jnp.int32),
            pltpu.VMEM((batch * span, DIM), jnp.float32),
            pltpu.VMEM_SHARED((NUM_SUBCORES * span, DIM), jnp.float32),
            pltpu.SemaphoreType.DMA((batch,)),
            pltpu.SemaphoreType.DMA,
            pltpu.SemaphoreType.DMA((2 * batch,)),
            pltpu.SemaphoreType.DMA((2 * batch,)),
            pltpu.SemaphoreType.DMA((2 * batch,)),
        ],
    )(input_ids, word_embeddings, position_embeddings)
    return out.reshape(batch, seq_len, DIM)
